# Initial kernel scaffold; baseline (speedup 1.0000x reference)
import jax
import jax.numpy as jnp
from jax.experimental import pallas as pl


def kernel(x, table):
    raise NotImplementedError("write your pallas kernel here")


# speedup vs baseline: 2.9733x; 2.9733x over previous
import functools

import jax
import jax.numpy as jnp
from jax import lax
from jax.experimental import pallas as pl
from jax.experimental.pallas import tpu as pltpu
from jax.experimental.pallas import tpu_sc as plsc

B = 4
L = 4096
D = 1024

_info = plsc.get_sparse_core_info()
_NC = _info.num_cores
_NS = _info.num_subcores
_NW = _NC * _NS
_ROWS = L // _NW
_CHUNK = 64
_NCH = _ROWS // _CHUNK

_mesh = plsc.VectorSubcoreMesh(core_axis_name="c", subcore_axis_name="s")


@functools.partial(
    pl.kernel,
    out_type=jax.ShapeDtypeStruct((B * L, D), jnp.float32),
    mesh=_mesh,
    scratch_types=[
        pltpu.VMEM((_CHUNK, D), jnp.float32),
        pltpu.SemaphoreType.DMA,
    ],
)
def _bcast_copy(table_hbm, out_hbm, buf, sem):
    wid = lax.axis_index("s") * _NC + lax.axis_index("c")
    base = wid * _ROWS
    for c in range(_NCH):
        off = base + c * _CHUNK
        pltpu.async_copy(table_hbm.at[pl.ds(off, _CHUNK)], buf, sem).wait()
        for b in range(B):
            pltpu.sync_copy(buf, out_hbm.at[pl.ds(b * L + off, _CHUNK)])


def kernel(x, table):
    del x
    out = _bcast_copy(table)
    return out.reshape(B, L, D)
